# Initial kernel scaffold; baseline (speedup 1.0000x reference)
import jax
import jax.numpy as jnp
from jax.experimental import pallas as pl


def kernel(user, item, neg_item, mf_user, mf_item, mlp_user, mlp_item, W1, b1, W2, b2, W3, b3, Wp, bp):
    raise NotImplementedError("write your pallas kernel here")


# speedup vs baseline: 1.4048x; 1.4048x over previous
import functools

import jax
import jax.numpy as jnp
from jax import lax
from jax.experimental import pallas as pl
from jax.experimental.pallas import tpu as pltpu
from jax.experimental.pallas import tpu_sc as plsc

_B = 16384
_DMF = 64
_DMLP = 128
_CH = 128


@functools.lru_cache(maxsize=1)
def _make_gather():
    info = plsc.get_sparse_core_info()
    nc, ns = info.num_cores, info.num_subcores
    nw = nc * ns
    bpw = _B // nw
    nch = bpw // _CH
    mesh = plsc.VectorSubcoreMesh(core_axis_name="c", subcore_axis_name="s")

    @functools.partial(
        pl.kernel,
        mesh=mesh,
        out_type=[jax.ShapeDtypeStruct((_B, _DMLP), jnp.float32)
                  for _ in range(6)],
        scratch_types=[
            pltpu.VMEM((nch, _CH), jnp.int32),
            pltpu.VMEM((bpw, _DMLP), jnp.float32),
            pltpu.SemaphoreType.DMA,
        ],
    )
    def gather(iu_mf, iu_mlp, ii_mf, ii_mlp, in_mf, in_mlp,
               mf_user2, mf_item2, mlp_user, mlp_item,
               o_umf, o_umlp, o_imf, o_imlp, o_nmf, o_nmlp,
               idx_v, buf, sem):
        wid = lax.axis_index("s") * nc + lax.axis_index("c")
        base = wid * bpw

        def one(idx2_hbm, table, out):
            pltpu.sync_copy(idx2_hbm.at[pl.ds(wid * nch, nch)], idx_v)
            cps = []
            for j in range(nch):
                cps.append(pltpu.async_copy(
                    table.at[idx_v.at[j]],
                    buf.at[pl.ds(j * _CH, _CH)], sem))
            for cp in cps:
                cp.wait()
            pltpu.sync_copy(buf, out.at[pl.ds(base, bpw)])

        one(iu_mf, mf_user2, o_umf)
        one(iu_mlp, mlp_user, o_umlp)
        one(ii_mf, mf_item2, o_imf)
        one(ii_mlp, mlp_item, o_imlp)
        one(in_mf, mf_item2, o_nmf)
        one(in_mlp, mlp_item, o_nmlp)

    return gather


def _half(wide, par):
    sel = par == 1
    return jnp.where(sel, wide[:, _DMF:], wide[:, :_DMF])


def _tc_body(umf_r, imf_r, nmf_r, umlp_r, imlp_r, nmlp_r,
             pu_r, pi_r, pn_r,
             w1a_r, w1b_r, b1_r, w2_r, b2_r, w3_r, b3_r,
             wpmf_r, wpmlp_r, bp_r, out_r):
    w1a = w1a_r[...]
    w1b = w1b_r[...]
    b1 = b1_r[...]
    w2 = w2_r[...]
    b2 = b2_r[...]
    w3 = w3_r[...]
    b3 = b3_r[...]
    wpmf = wpmf_r[...]
    wpmlp = wpmlp_r[...]
    bp = bp_r[...]
    umf = _half(umf_r[...], pu_r[...])
    umlp = umlp_r[...]
    u1 = jnp.dot(umlp, w1a, preferred_element_type=jnp.float32)

    def score(imf, imlp):
        h = jnp.maximum(
            u1 + jnp.dot(imlp, w1b, preferred_element_type=jnp.float32) + b1,
            0.0)
        h = jnp.maximum(
            jnp.dot(h, w2, preferred_element_type=jnp.float32) + b2, 0.0)
        h = jnp.maximum(
            jnp.dot(h, w3, preferred_element_type=jnp.float32) + b3, 0.0)
        logit = (jnp.sum(umf * imf * wpmf, axis=1, keepdims=True)
                 + jnp.sum(h * wpmlp, axis=1, keepdims=True) + bp)
        return jax.nn.sigmoid(logit)

    ps = score(_half(imf_r[...], pi_r[...]), imlp_r[...])
    ns = score(_half(nmf_r[...], pn_r[...]), nmlp_r[...])
    part = jnp.sum(jax.nn.softplus(ns - ps)) * (1.0 / _B)

    @pl.when(pl.program_id(0) == 0)
    def _():
        out_r[...] = jnp.zeros_like(out_r)

    out_r[...] += part


def _tc_loss(umf, imf, nmf, umlp, imlp, nmlp, pu, pi, pn,
             w1a, w1b, b1, w2, b2, w3, b3, wpmf, wpmlp, bp, *,
             interpret=False):
    bb = 2048
    nb = _B // bb

    def fixed(shape):
        return pl.BlockSpec(shape, lambda i: (0, 0))

    def batched(d):
        return pl.BlockSpec((bb, d), lambda i: (i, 0))

    return pl.pallas_call(
        _tc_body,
        grid=(nb,),
        in_specs=[
            batched(_DMLP), batched(_DMLP), batched(_DMLP),
            batched(_DMLP), batched(_DMLP), batched(_DMLP),
            batched(1), batched(1), batched(1),
            fixed((_DMLP, _DMLP)), fixed((_DMLP, _DMLP)), fixed((1, _DMLP)),
            fixed((_DMLP, 64)), fixed((1, 64)),
            fixed((64, 32)), fixed((1, 32)),
            fixed((1, _DMF)), fixed((1, 32)), fixed((1, 1)),
        ],
        out_specs=pl.BlockSpec((1, 1), lambda i: (0, 0)),
        out_shape=jax.ShapeDtypeStruct((1, 1), jnp.float32),
        compiler_params=pltpu.CompilerParams(
            dimension_semantics=("arbitrary",)),
        interpret=interpret,
    )(umf, imf, nmf, umlp, imlp, nmlp, pu, pi, pn,
      w1a, w1b, b1, w2, b2, w3, b3, wpmf, wpmlp, bp)


def kernel(user, item, neg_item, mf_user, mf_item, mlp_user, mlp_item,
           W1, b1, W2, b2, W3, b3, Wp, bp):
    user = user.astype(jnp.int32)
    item = item.astype(jnp.int32)
    neg_item = neg_item.astype(jnp.int32)
    nix = _B // _CH

    def prep(idx):
        return ((idx >> 1).reshape(nix, _CH), idx.reshape(nix, _CH),
                (idx & 1).reshape(_B, 1))

    iu_mf, iu_mlp, pu = prep(user)
    ii_mf, ii_mlp, pi = prep(item)
    in_mf, in_mlp, pn = prep(neg_item)
    mf_user2 = mf_user.reshape(-1, _DMLP)
    mf_item2 = mf_item.reshape(-1, _DMLP)
    umf, umlp, imf, imlp, nmf, nmlp = _make_gather()(
        iu_mf, iu_mlp, ii_mf, ii_mlp, in_mf, in_mlp,
        mf_user2, mf_item2, mlp_user, mlp_item)
    w1a = W1[:_DMLP]
    w1b = W1[_DMLP:]
    wp = Wp.reshape(1, _DMF + 32)
    out = _tc_loss(
        umf, imf, nmf, umlp, imlp, nmlp, pu, pi, pn,
        w1a, w1b, b1.reshape(1, _DMLP), W2, b2.reshape(1, 64),
        W3, b3.reshape(1, 32), wp[:, :_DMF], wp[:, _DMF:], bp.reshape(1, 1))
    return out[0, 0]


# speedup vs baseline: 1.4299x; 1.0179x over previous
import functools

import jax
import jax.numpy as jnp
from jax import lax
from jax.experimental import pallas as pl
from jax.experimental.pallas import tpu as pltpu
from jax.experimental.pallas import tpu_sc as plsc

_B = 16384
_DMF = 64
_DMLP = 128
_CH = 128


_NSETS = 6
_DEPTH = 4
_LAG_W = 2


@functools.lru_cache(maxsize=1)
def _make_gather():
    info = plsc.get_sparse_core_info()
    nc, ns = info.num_cores, info.num_subcores
    nw = nc * ns
    bpw = _B // nw
    nch = bpw // _CH
    nk = _NSETS * nch
    mesh = plsc.VectorSubcoreMesh(core_axis_name="c", subcore_axis_name="s")

    @functools.partial(
        pl.kernel,
        mesh=mesh,
        out_type=[jax.ShapeDtypeStruct((_B, _DMLP), jnp.float32)
                  for _ in range(_NSETS)],
        scratch_types=[
            pltpu.VMEM((nk, _CH), jnp.int32),
        ] + [pltpu.VMEM((_CH, _DMLP), jnp.float32) for _ in range(_DEPTH)] + [
            pltpu.SemaphoreType.DMA,
            pltpu.SemaphoreType.DMA,
        ],
    )
    def gather(idx_all, mf_user2, mf_item2, mlp_user, mlp_item,
               o_umf, o_umlp, o_imf, o_imlp, o_nmf, o_nmlp,
               idx_v, *bufs_and_sems):
        bufs = bufs_and_sems[:_DEPTH]
        sem_g, sem_w = bufs_and_sems[_DEPTH], bufs_and_sems[_DEPTH + 1]
        tables = [mf_user2, mlp_user, mf_item2, mlp_item, mf_item2, mlp_item]
        outs = [o_umf, o_umlp, o_imf, o_imlp, o_nmf, o_nmlp]
        wid = lax.axis_index("s") * nc + lax.axis_index("c")
        base = wid * bpw
        pltpu.sync_copy(idx_all.at[wid], idx_v)

        gcps = [None] * nk
        wcps = [None] * nk

        def issue_write(m):
            s, j = divmod(m, nch)
            gcps[m].wait()
            wcps[m] = pltpu.async_copy(
                bufs[m % _DEPTH],
                outs[s].at[pl.ds(base + j * _CH, _CH)], sem_w)

        for k in range(nk):
            if k >= _LAG_W:
                issue_write(k - _LAG_W)
            if k >= _DEPTH:
                wcps[k - _DEPTH].wait()
            s = k // nch
            gcps[k] = pltpu.async_copy(
                tables[s].at[idx_v.at[k]], bufs[k % _DEPTH], sem_g)
        for m in range(nk - _LAG_W, nk):
            issue_write(m)
        for m in range(nk - _DEPTH, nk):
            wcps[m].wait()

    return gather


def _half(wide, par):
    sel = par == 1
    return jnp.where(sel, wide[:, _DMF:], wide[:, :_DMF])


def _tc_body(umf_r, imf_r, nmf_r, umlp_r, imlp_r, nmlp_r,
             pu_r, pi_r, pn_r,
             w1a_r, w1b_r, b1_r, w2_r, b2_r, w3_r, b3_r,
             wpmf_r, wpmlp_r, bp_r, out_r):
    w1a = w1a_r[...]
    w1b = w1b_r[...]
    b1 = b1_r[...]
    w2 = w2_r[...]
    b2 = b2_r[...]
    w3 = w3_r[...]
    b3 = b3_r[...]
    wpmf = wpmf_r[...]
    wpmlp = wpmlp_r[...]
    bp = bp_r[...]
    umf = _half(umf_r[...], pu_r[...])
    umlp = umlp_r[...]
    u1 = jnp.dot(umlp, w1a, preferred_element_type=jnp.float32)

    def score(imf, imlp):
        h = jnp.maximum(
            u1 + jnp.dot(imlp, w1b, preferred_element_type=jnp.float32) + b1,
            0.0)
        h = jnp.maximum(
            jnp.dot(h, w2, preferred_element_type=jnp.float32) + b2, 0.0)
        h = jnp.maximum(
            jnp.dot(h, w3, preferred_element_type=jnp.float32) + b3, 0.0)
        logit = (jnp.sum(umf * imf * wpmf, axis=1, keepdims=True)
                 + jnp.sum(h * wpmlp, axis=1, keepdims=True) + bp)
        return jax.nn.sigmoid(logit)

    ps = score(_half(imf_r[...], pi_r[...]), imlp_r[...])
    ns = score(_half(nmf_r[...], pn_r[...]), nmlp_r[...])
    part = jnp.sum(jax.nn.softplus(ns - ps)) * (1.0 / _B)

    @pl.when(pl.program_id(0) == 0)
    def _():
        out_r[...] = jnp.zeros_like(out_r)

    out_r[...] += part


def _tc_loss(umf, imf, nmf, umlp, imlp, nmlp, pu, pi, pn,
             w1a, w1b, b1, w2, b2, w3, b3, wpmf, wpmlp, bp, *,
             interpret=False):
    bb = 2048
    nb = _B // bb

    def fixed(shape):
        return pl.BlockSpec(shape, lambda i: (0, 0))

    def batched(d):
        return pl.BlockSpec((bb, d), lambda i: (i, 0))

    return pl.pallas_call(
        _tc_body,
        grid=(nb,),
        in_specs=[
            batched(_DMLP), batched(_DMLP), batched(_DMLP),
            batched(_DMLP), batched(_DMLP), batched(_DMLP),
            batched(1), batched(1), batched(1),
            fixed((_DMLP, _DMLP)), fixed((_DMLP, _DMLP)), fixed((1, _DMLP)),
            fixed((_DMLP, 64)), fixed((1, 64)),
            fixed((64, 32)), fixed((1, 32)),
            fixed((1, _DMF)), fixed((1, 32)), fixed((1, 1)),
        ],
        out_specs=pl.BlockSpec((1, 1), lambda i: (0, 0)),
        out_shape=jax.ShapeDtypeStruct((1, 1), jnp.float32),
        compiler_params=pltpu.CompilerParams(
            dimension_semantics=("arbitrary",)),
        interpret=interpret,
    )(umf, imf, nmf, umlp, imlp, nmlp, pu, pi, pn,
      w1a, w1b, b1, w2, b2, w3, b3, wpmf, wpmlp, bp)


def kernel(user, item, neg_item, mf_user, mf_item, mlp_user, mlp_item,
           W1, b1, W2, b2, W3, b3, Wp, bp):
    user = user.astype(jnp.int32)
    item = item.astype(jnp.int32)
    neg_item = neg_item.astype(jnp.int32)
    stack = jnp.stack([user >> 1, user, item >> 1, item,
                       neg_item >> 1, neg_item])
    nw = 32
    bpw = _B // nw
    idx_all = (stack.reshape(_NSETS, nw, bpw // _CH, _CH)
               .transpose(1, 0, 2, 3)
               .reshape(nw, _NSETS * (bpw // _CH), _CH))
    pu = (user & 1).reshape(_B, 1)
    pi = (item & 1).reshape(_B, 1)
    pn = (neg_item & 1).reshape(_B, 1)
    mf_user2 = mf_user.reshape(-1, _DMLP)
    mf_item2 = mf_item.reshape(-1, _DMLP)
    umf, umlp, imf, imlp, nmf, nmlp = _make_gather()(
        idx_all, mf_user2, mf_item2, mlp_user, mlp_item)
    w1a = W1[:_DMLP]
    w1b = W1[_DMLP:]
    wp = Wp.reshape(1, _DMF + 32)
    out = _tc_loss(
        umf, imf, nmf, umlp, imlp, nmlp, pu, pi, pn,
        w1a, w1b, b1.reshape(1, _DMLP), W2, b2.reshape(1, 64),
        W3, b3.reshape(1, 32), wp[:, :_DMF], wp[:, _DMF:], bp.reshape(1, 1))
    return out[0, 0]
